# Initial kernel scaffold; baseline (speedup 1.0000x reference)
import jax
import jax.numpy as jnp
from jax.experimental import pallas as pl


def kernel(idx, wte, wpe, reasoning_prompts):
    raise NotImplementedError("write your pallas kernel here")


# speedup vs baseline: 1.2754x; 1.2754x over previous
import functools

import jax
import jax.numpy as jnp
from jax import lax
from jax.experimental import pallas as pl
from jax.experimental.pallas import tpu as pltpu
from jax.experimental.pallas import tpu_sc as plsc

B = 4
T = 2048
D = 128
NP = 8
NW = 32
CHUNK = (B * T) // NW
WPB = T // CHUNK
NGATHER = CHUNK // 128
LANES = 16


def _emb_body(idx_hbm, wte_hbm, wpe_hbm, prompts_hbm, out_hbm,
              idx_v, rows_v, wpe_v, prompts_v, gsem):
    cid = lax.axis_index("c")
    sid = lax.axis_index("s")
    wid = sid * 2 + cid
    b = wid // WPB
    t0 = (wid % WPB) * CHUNK

    for k in range(NGATHER):
        pltpu.sync_copy(idx_hbm.at[b, pl.ds(t0 + k * 128, 128)], idx_v.at[k])

    copies = [
        pltpu.async_copy(wte_hbm.at[idx_v.at[k]],
                         rows_v.at[pl.ds(k * 128, 128)], gsem)
        for k in range(NGATHER)
    ]
    pltpu.sync_copy(wpe_hbm.at[pl.ds(t0, CHUNK)], wpe_v)
    for c in copies:
        c.wait()

    def add_row(i, _):
        for j in range(D // LANES):
            s = pl.ds(j * LANES, LANES)
            rows_v[i, s] = rows_v[i, s] + wpe_v[i, s]
        return _

    lax.fori_loop(0, CHUNK, add_row, None)

    pltpu.sync_copy(rows_v, out_hbm.at[b, pl.ds(NP + t0, CHUNK)])

    @pl.when(t0 == 0)
    def _():
        pltpu.sync_copy(prompts_hbm, prompts_v)
        pltpu.sync_copy(prompts_v, out_hbm.at[b, pl.ds(0, NP)])


@jax.jit
def kernel(idx, wte, wpe, reasoning_prompts):
    mesh = plsc.VectorSubcoreMesh(core_axis_name="c", subcore_axis_name="s")
    run = functools.partial(
        pl.kernel,
        out_type=jax.ShapeDtypeStruct((B, NP + T, D), jnp.float32),
        mesh=mesh,
        scratch_types=[
            pltpu.VMEM((NGATHER, 128), jnp.int32),
            pltpu.VMEM((CHUNK, D), jnp.float32),
            pltpu.VMEM((CHUNK, D), jnp.float32),
            pltpu.VMEM((NP, D), jnp.float32),
            pltpu.SemaphoreType.DMA,
        ],
    )(_emb_body)
    return run(idx.astype(jnp.int32), wte, wpe, reasoning_prompts)


# speedup vs baseline: 1.4078x; 1.1038x over previous
import functools

import jax
import jax.numpy as jnp
from jax import lax
from jax.experimental import pallas as pl
from jax.experimental.pallas import tpu as pltpu
from jax.experimental.pallas import tpu_sc as plsc

B = 4
T = 2048
D = 128
NP = 8
NW = 32
TCHUNK = T // NW
LANES = 16


def _emb_body(idx_hbm, wte_hbm, wpe_hbm, prompts_hbm, out_hbm,
              idx_v, rows_v, wpe_v, prompts_v, isem, gsem, ssem):
    cid = lax.axis_index("c")
    sid = lax.axis_index("s")
    wid = sid * 2 + cid
    t0 = wid * TCHUNK

    icopies = [
        pltpu.async_copy(idx_hbm.at[b, pl.ds(t0, TCHUNK)], idx_v.at[b], isem)
        for b in range(B)
    ]
    for c in icopies:
        c.wait()

    gcopies = [
        pltpu.async_copy(wte_hbm.at[idx_v.at[b]], rows_v.at[b], gsem)
        for b in range(B)
    ]
    pltpu.sync_copy(wpe_hbm.at[pl.ds(t0, TCHUNK)], wpe_v)

    @pl.when(wid < B)
    def _():
        pltpu.sync_copy(prompts_hbm, prompts_v)
        pltpu.sync_copy(prompts_v, out_hbm.at[wid, pl.ds(0, NP)])

    scopies = []
    for b in range(B):
        gcopies[b].wait()

        def add_row(i, _, b=b):
            for j in range(D // LANES):
                s = pl.ds(j * LANES, LANES)
                plsc.addupdate(rows_v.at[b, i, s], wpe_v[i, s])
            return _

        lax.fori_loop(0, TCHUNK, add_row, None)
        scopies.append(
            pltpu.async_copy(rows_v.at[b],
                             out_hbm.at[b, pl.ds(NP + t0, TCHUNK)], ssem))
    for c in scopies:
        c.wait()


@jax.jit
def kernel(idx, wte, wpe, reasoning_prompts):
    mesh = plsc.VectorSubcoreMesh(core_axis_name="c", subcore_axis_name="s")
    run = functools.partial(
        pl.kernel,
        out_type=jax.ShapeDtypeStruct((B, NP + T, D), jnp.float32),
        mesh=mesh,
        scratch_types=[
            pltpu.VMEM((B, TCHUNK), jnp.int32),
            pltpu.VMEM((B, TCHUNK, D), jnp.float32),
            pltpu.VMEM((TCHUNK, D), jnp.float32),
            pltpu.VMEM((NP, D), jnp.float32),
            pltpu.SemaphoreType.DMA,
            pltpu.SemaphoreType.DMA,
            pltpu.SemaphoreType.DMA,
        ],
    )(_emb_body)
    return run(idx.astype(jnp.int32), wte, wpe, reasoning_prompts)
